# baseline (speedup 1.0000x reference)
import functools

import jax
import jax.numpy as jnp
from jax.experimental import pallas as pl
from jax.experimental.pallas import tpu as pltpu

B = 128
V = 100000
CHUNK = 2048
NCHUNK = (V + CHUNK - 1) // CHUNK

_KS0 = 0
_KS1 = 42
_KS2 = _KS0 ^ _KS1 ^ 0x1BD11BDA

_ROT0 = (13, 15, 26, 6)
_ROT1 = (17, 29, 16, 24)
_TINY = 1.1754943508222875e-38
_NEG_INF = float("-inf")


def _rotl(x, r):
    return (x << jnp.uint32(r)) | (x >> jnp.uint32(32 - r))


def _threefry_bits(counts):
    ks0 = jnp.uint32(_KS0)
    ks1 = jnp.uint32(_KS1)
    ks2 = jnp.uint32(_KS2)
    x0 = jnp.full(counts.shape, ks0, jnp.uint32)
    x1 = counts + ks1
    sched = ((_ROT0, ks1, ks2, 1), (_ROT1, ks2, ks0, 2), (_ROT0, ks0, ks1, 3),
             (_ROT1, ks1, ks2, 4), (_ROT0, ks2, ks0, 5))
    for rots, ka, kb, inc in sched:
        for r in rots:
            x0 = x0 + x1
            x1 = _rotl(x1, r)
            x1 = x1 ^ x0
        x0 = x0 + ka
        x1 = x1 + kb + jnp.uint32(inc)
    return x0 ^ x1


def _sampler_kernel(x_ref, out_ref, mval_ref):
    j = pl.program_id(0)
    base = j * CHUNK

    @pl.when(j == 0)
    def _init():
        mval_ref[...] = jnp.full((B, 1), jnp.float32(_NEG_INF), jnp.float32)
        out_ref[...] = jnp.zeros((B, 1), jnp.int32)

    row = jax.lax.broadcasted_iota(jnp.int32, (B, CHUNK), 0)
    col = jax.lax.broadcasted_iota(jnp.int32, (B, CHUNK), 1)
    vcol = base + col
    flat = (row * V + vcol).astype(jnp.uint32)

    bits = _threefry_bits(flat)
    mant = (bits >> jnp.uint32(9)) | jnp.uint32(0x3F800000)
    f = jax.lax.bitcast_convert_type(mant, jnp.float32) - jnp.float32(1.0)
    u = jnp.maximum(f, jnp.float32(_TINY))
    g = -jnp.log(-jnp.log(u))

    score = x_ref[...] + g
    score = jnp.where(vcol < V, score, jnp.float32(_NEG_INF))

    lmax = jnp.max(score, axis=1, keepdims=True)
    lidx = jnp.min(jnp.where(score == lmax, vcol, jnp.int32(V)), axis=1,
                   keepdims=True)

    upd = lmax > mval_ref[...]
    mval_ref[...] = jnp.where(upd, lmax, mval_ref[...])
    out_ref[...] = jnp.where(upd, lidx, out_ref[...])


@jax.jit
def kernel(input):
    out = pl.pallas_call(
        _sampler_kernel,
        grid=(NCHUNK,),
        in_specs=[pl.BlockSpec((B, CHUNK), lambda j: (0, j))],
        out_specs=pl.BlockSpec((B, 1), lambda j: (0, 0)),
        out_shape=jax.ShapeDtypeStruct((B, 1), jnp.int32),
        scratch_shapes=[pltpu.VMEM((B, 1), jnp.float32)],
    )(input)
    return out


# speedup vs baseline: 1.2020x; 1.2020x over previous
import jax
import jax.numpy as jnp
from jax.experimental import pallas as pl
from jax.experimental.pallas import tpu as pltpu

B = 128
V = 100000
CHUNK = 2048
NCHUNK = (V + CHUNK - 1) // CHUNK

_KS0 = 0
_KS1 = 42
_KS2 = _KS0 ^ _KS1 ^ 0x1BD11BDA
_ROT0 = (13, 15, 26, 6)
_ROT1 = (17, 29, 16, 24)
_TINY = 1.1754943508222875e-38
_NEG_INF = float("-inf")


def _rotl(x, r):
    return (x << jnp.uint32(r)) | (x >> jnp.uint32(32 - r))


def _threefry_bits(counts):
    ks0 = jnp.uint32(_KS0)
    ks1 = jnp.uint32(_KS1)
    ks2 = jnp.uint32(_KS2)
    x0 = jnp.full(counts.shape, ks0, jnp.uint32)
    x1 = counts + ks1
    sched = ((_ROT0, ks1, ks2, 1), (_ROT1, ks2, ks0, 2), (_ROT0, ks0, ks1, 3),
             (_ROT1, ks1, ks2, 4), (_ROT0, ks2, ks0, 5))
    for rots, ka, kb, inc in sched:
        for r in rots:
            x0 = x0 + x1
            x1 = _rotl(x1, r)
            x1 = x1 ^ x0
        x0 = x0 + ka
        x1 = x1 + kb + jnp.uint32(inc)
    return x0 ^ x1


SUB = 128


def _partial_kernel(xt_ref, max_ref, idx_ref):
    j = pl.program_id(0)
    base = j * CHUNK
    m_acc = jnp.full((1, B), jnp.float32(_NEG_INF), jnp.float32)
    i_acc = jnp.full((1, B), jnp.int32(V), jnp.int32)
    for s in range(CHUNK // SUB):
        off = s * SUB
        vrow = (base + off) + jax.lax.broadcasted_iota(jnp.int32, (SUB, B), 0)
        lane = jax.lax.broadcasted_iota(jnp.int32, (SUB, B), 1)
        flat = (lane * V + vrow).astype(jnp.uint32)

        bits = _threefry_bits(flat)
        mant = (bits >> jnp.uint32(9)) | jnp.uint32(0x3F800000)
        f = jax.lax.bitcast_convert_type(mant, jnp.float32) - jnp.float32(1.0)
        u = jnp.maximum(f, jnp.float32(_TINY))
        g = -jnp.log(-jnp.log(u))

        score = xt_ref[pl.ds(off, SUB), :] + g
        score = jnp.where(vrow < V, score, jnp.float32(_NEG_INF))

        smax = jnp.max(score, axis=0, keepdims=True)
        sidx = jnp.min(jnp.where(score == smax, vrow, jnp.int32(V)), axis=0,
                       keepdims=True)
        upd = smax > m_acc
        m_acc = jnp.where(upd, smax, m_acc)
        i_acc = jnp.where(upd, sidx, i_acc)
    max_ref[...] = m_acc[None]
    idx_ref[...] = i_acc[None]


def _merge_kernel(max_ref, idx_ref, out_ref):
    m = max_ref[...]
    gmax = jnp.max(m, axis=0, keepdims=True)
    out_ref[...] = jnp.min(
        jnp.where(m == gmax, idx_ref[...], jnp.int32(V)), axis=0)


@jax.jit
def kernel(input):
    xt = input.T
    maxs, idxs = pl.pallas_call(
        _partial_kernel,
        grid=(NCHUNK,),
        in_specs=[pl.BlockSpec((CHUNK, B), lambda j: (j, 0))],
        out_specs=[pl.BlockSpec((1, 1, B), lambda j: (j, 0, 0)),
                   pl.BlockSpec((1, 1, B), lambda j: (j, 0, 0))],
        out_shape=[jax.ShapeDtypeStruct((NCHUNK, 1, B), jnp.float32),
                   jax.ShapeDtypeStruct((NCHUNK, 1, B), jnp.int32)],
        compiler_params=pltpu.CompilerParams(
            dimension_semantics=("parallel",)),
    )(xt)
    out = pl.pallas_call(
        _merge_kernel,
        out_shape=jax.ShapeDtypeStruct((1, B), jnp.int32),
    )(maxs, idxs)
    return out.reshape(B, 1)


# speedup vs baseline: 1.3136x; 1.0929x over previous
import jax
import jax.numpy as jnp
from jax.experimental import pallas as pl
from jax.experimental.pallas import tpu as pltpu

B = 128
V = 100000
NCHUNK = 25
CHUNK = V // NCHUNK
SUB = 80
NSLICE = CHUNK // SUB

_KS0 = 0
_KS1 = 42
_KS2 = _KS0 ^ _KS1 ^ 0x1BD11BDA
_ROT0 = (13, 15, 26, 6)
_ROT1 = (17, 29, 16, 24)
_TINY = 1.1754943508222875e-38
_NEG_INF = float("-inf")


def _rotl(x, r):
    return (x << jnp.uint32(r)) | (x >> jnp.uint32(32 - r))


def _threefry_bits(x1):
    ks0 = jnp.uint32(_KS0)
    ks1 = jnp.uint32(_KS1)
    ks2 = jnp.uint32(_KS2)
    x0 = jnp.full(x1.shape, ks0, jnp.uint32)
    sched = ((_ROT0, ks1, ks2, 1), (_ROT1, ks2, ks0, 2), (_ROT0, ks0, ks1, 3),
             (_ROT1, ks1, ks2, 4), (_ROT0, ks2, ks0, 5))
    for rots, ka, kb, inc in sched:
        for r in rots:
            x0 = x0 + x1
            x1 = _rotl(x1, r)
            x1 = x1 ^ x0
        x0 = x0 + ka
        x1 = x1 + kb + jnp.uint32(inc)
    return x0 ^ x1


def _neg_gumbel(x1):
    bits = _threefry_bits(x1)
    mant = (bits >> jnp.uint32(9)) | jnp.uint32(0x3F800000)
    f = jax.lax.bitcast_convert_type(mant, jnp.float32) - jnp.float32(1.0)
    u = jnp.maximum(f, jnp.float32(_TINY))
    return jnp.log(-jnp.log(u))


def _partial_kernel(xt_ref, max_ref, idx_ref):
    j = pl.program_id(0)
    base = j * CHUNK
    subl = jax.lax.broadcasted_iota(jnp.int32, (SUB, B), 0)
    lane = jax.lax.broadcasted_iota(jnp.int32, (SUB, B), 1)
    lvi = (lane * V + subl).astype(jnp.uint32)

    macc = jnp.full((SUB, B), jnp.float32(_NEG_INF), jnp.float32)
    sacc = jnp.zeros((SUB, B), jnp.int32)
    for s in range(NSLICE):
        off = s * SUB
        ng = _neg_gumbel(lvi + jnp.uint32(base + off + _KS1))
        score = xt_ref[pl.ds(off, SUB), :] - ng
        upd = score > macc
        macc = jnp.where(upd, score, macc)
        sacc = jnp.where(upd, s, sacc)

    vidx = base + sacc * SUB + subl
    cmax = jnp.max(macc, axis=0, keepdims=True)
    cidx = jnp.min(jnp.where(macc == cmax, vidx, jnp.int32(V)), axis=0,
                   keepdims=True)
    max_ref[...] = cmax[None]
    idx_ref[...] = cidx[None]


def _merge_kernel(max_ref, idx_ref, out_ref):
    m = max_ref[...]
    gmax = jnp.max(m, axis=0, keepdims=True)
    out_ref[...] = jnp.min(
        jnp.where(m == gmax, idx_ref[...], jnp.int32(V)), axis=0)


@jax.jit
def kernel(input):
    xt = input.T
    maxs, idxs = pl.pallas_call(
        _partial_kernel,
        grid=(NCHUNK,),
        in_specs=[pl.BlockSpec((CHUNK, B), lambda j: (j, 0))],
        out_specs=[pl.BlockSpec((1, 1, B), lambda j: (j, 0, 0)),
                   pl.BlockSpec((1, 1, B), lambda j: (j, 0, 0))],
        out_shape=[jax.ShapeDtypeStruct((NCHUNK, 1, B), jnp.float32),
                   jax.ShapeDtypeStruct((NCHUNK, 1, B), jnp.int32)],
        compiler_params=pltpu.CompilerParams(
            dimension_semantics=("parallel",)),
    )(xt)
    out = pl.pallas_call(
        _merge_kernel,
        out_shape=jax.ShapeDtypeStruct((1, B), jnp.int32),
    )(maxs, idxs)
    return out.reshape(B, 1)
